# baseline (speedup 1.0000x reference)
import functools

import jax
import jax.numpy as jnp
from jax import lax
from jax.experimental import pallas as pl
from jax.experimental.pallas import tpu as pltpu
from jax.experimental.pallas import tpu_sc as plsc

SPAD = 208
NC, NS = 2, 16
NW = NC * NS
REG = 32768
SLAB = 8192


def _attn_call(dh, enc, ids, wa, wb, ba, wv, wgh, wgc, bg):
    B, S, H = enc.shape
    BB = 8

    def body(dh_ref, enc_ref, ids_ref, wa_ref, wb_ref, ba_ref, wv_ref,
             wgh_ref, wgc_ref, bg_ref,
             cw_ref, gate_ref, upd_ref, idsf_ref):
        pid = pl.program_id(0)
        dhb = dh_ref[...]
        encb = enc_ref[...]
        dpart = jnp.dot(dhb, wa_ref[...], preferred_element_type=jnp.float32)
        dpart = dpart + ba_ref[...]
        e = jnp.dot(encb.reshape(BB * S, H), wb_ref[...],
                    preferred_element_type=jnp.float32)
        e = jnp.tanh(e.reshape(BB, S, H) + dpart[:, None, :])
        sc = jnp.sum(e * wv_ref[...][None, :, :], axis=2)
        m = jnp.max(sc, axis=1, keepdims=True)
        ex = jnp.exp(sc - m)
        w = ex / jnp.sum(ex, axis=1, keepdims=True)
        cw_ref[...] = w
        ctx = jnp.sum(w[:, :, None] * encb, axis=1)
        g = jnp.dot(dhb, wgh_ref[...], preferred_element_type=jnp.float32)
        g = g + jnp.dot(ctx, wgc_ref[...], preferred_element_type=jnp.float32)
        g = jax.nn.sigmoid(g + bg_ref[...])
        gate_ref[...] = g
        idsb = ids_ref[...]
        ids_pad = jnp.concatenate([idsb, idsb[:, :SPAD - S]], axis=1)
        w_pad = jnp.concatenate(
            [w, jnp.zeros((BB, SPAD - S), jnp.float32)], axis=1)
        upd_ref[...] = w_pad * g
        row = pid * BB + lax.broadcasted_iota(jnp.int32, (BB, 1), 0)
        nbt = B // 128
        idsf_ref[...] = (((ids_pad >> 3) * nbt + (row >> 7)) << 10) \
            + ((ids_pad & 7) << 7) + (row & 127)

    return pl.pallas_call(
        body,
        grid=(B // BB,),
        in_specs=[
            pl.BlockSpec((BB, H), lambda i: (i, 0)),
            pl.BlockSpec((BB, S, H), lambda i: (i, 0, 0)),
            pl.BlockSpec((BB, S), lambda i: (i, 0)),
            pl.BlockSpec((H, H), lambda i: (0, 0)),
            pl.BlockSpec((H, H), lambda i: (0, 0)),
            pl.BlockSpec((1, H), lambda i: (0, 0)),
            pl.BlockSpec((1, H), lambda i: (0, 0)),
            pl.BlockSpec((H, 1), lambda i: (0, 0)),
            pl.BlockSpec((H, 1), lambda i: (0, 0)),
            pl.BlockSpec((1, 1), lambda i: (0, 0)),
        ],
        out_specs=[
            pl.BlockSpec((BB, S), lambda i: (i, 0)),
            pl.BlockSpec((BB, 1), lambda i: (i, 0)),
            pl.BlockSpec((BB, SPAD), lambda i: (i, 0)),
            pl.BlockSpec((BB, SPAD), lambda i: (i, 0)),
        ],
        out_shape=[
            jax.ShapeDtypeStruct((B, S), jnp.float32),
            jax.ShapeDtypeStruct((B, 1), jnp.float32),
            jax.ShapeDtypeStruct((B, SPAD), jnp.float32),
            jax.ShapeDtypeStruct((B, SPAD), jnp.int32),
        ],
    )(dh, enc, ids, wa, wb, ba, wv, wgh, wgc, bg)


def _sc_stream_call(vocab_lin, idsf2, upd2, gate2):
    TOT = vocab_lin.shape[0]
    GB = gate2.shape[0]
    Bv = GB * 128
    slab = 8 * Bv
    cmask = slab // 16 - 1
    REGB = 19
    reg = 1 << REGB
    nfull = TOT // reg
    tail = TOT - nfull * reg
    NRWS = idsf2.shape[0]
    TR = NRWS // NS
    r0 = (nfull + 1) // 2
    mesh = plsc.VectorSubcoreMesh(core_axis_name="c", subcore_axis_name="s")

    @functools.partial(
        pl.kernel,
        out_type=jax.ShapeDtypeStruct((TOT,), jnp.float32),
        mesh=mesh,
        scratch_types=[
            pltpu.VMEM((GB, 128), jnp.float32),
            pltpu.VMEM((slab,), jnp.float32),
            pltpu.VMEM((TR, 128), jnp.int32),
            pltpu.VMEM((TR, 128), jnp.float32),
            pltpu.VMEM((TR, 128), jnp.int32),
            pltpu.VMEM((TR, 128), jnp.float32),
            pltpu.VMEM((reg // NS,), jnp.float32),
            pltpu.VMEM_SHARED((reg + 128,), jnp.float32),
            pltpu.SemaphoreType.DMA,
        ],
    )
    def sc_sweep(vocab_ref, idsf_ref, upd_ref, gate_ref, out_ref,
                 gbuf, omg, fidx, fupd, sidx, sval, sbuf, shared, sem):
        cid = lax.axis_index("c")
        sid = lax.axis_index("s")

        pltpu.sync_copy(gate_ref, gbuf)

        def exp_body(c, carry):
            src = gbuf[c >> 6, pl.ds((c & 7) * 16, 16)]
            omg[pl.ds(c * 16, 16)] = 1.0 - src
            return carry

        lax.fori_loop(0, slab // 16, exp_body, 0)

        pltpu.sync_copy(idsf_ref.at[pl.ds(sid * TR, TR)], fidx)
        pltpu.sync_copy(upd_ref.at[pl.ds(sid * TR, TR)], fupd)

        def do_region(r, SLICE):
            base = r << REGB
            my_lo = sid * SLICE
            sb = sbuf.at[pl.ds(0, SLICE)]
            src0 = pl.multiple_of(base + my_lo, 1024)
            pltpu.sync_copy(vocab_ref.at[pl.ds(src0, SLICE)], sb)
            p0 = (my_lo >> 4) & cmask

            def scale_body(c, carry):
                sl = pl.ds(pl.multiple_of(c * 16, 16), 16)
                pat = pl.ds(pl.multiple_of((((p0 + c) & cmask) * 16), 16), 16)
                sbuf[sl] = sbuf[sl] * omg[pat]
                return carry

            lax.fori_loop(0, SLICE // 16, scale_body, 0)
            mlo = pl.multiple_of(my_lo, 1024)
            pltpu.sync_copy(sb, shared.at[pl.ds(mlo, SLICE)])
            plsc.subcore_barrier()

            rv = jnp.full((16,), r, jnp.int32)

            def sel_body(c, carry):
                row = c >> 3
                sl = pl.ds((c & 7) * 16, 16)
                iv = fidx[row, sl]
                uv = fupd[row, sl]
                msk = (iv >> REGB) == rv
                sidx[row, sl] = jnp.where(msk, iv & (reg - 1),
                                          reg + (iv & 127))
                sval[row, sl] = jnp.where(msk, uv, 0.0)
                return carry

            lax.fori_loop(0, TR * 8, sel_body, 0)

            def fire_body(j, carry):
                pltpu.async_copy(sval.at[j], shared.at[sidx.at[j]], sem,
                                 add=True)
                return carry

            lax.fori_loop(0, TR, fire_body, 0)
            pltpu.make_async_copy(upd_ref.at[pl.ds(0, TR)], sval, sem).wait()
            plsc.subcore_barrier()
            pltpu.sync_copy(shared.at[pl.ds(mlo, SLICE)], sb)
            pltpu.sync_copy(sb, out_ref.at[pl.ds(src0, SLICE)])
            plsc.subcore_barrier()

        n_my = jnp.where(cid == 0, r0, nfull - r0)

        def region_body(j, carry):
            do_region(jnp.where(cid == 0, j, r0 + j), reg // NS)
            return carry

        lax.fori_loop(0, n_my, region_body, 0)

        @pl.when(jnp.logical_and(cid == 1, tail > 0))
        def _():
            do_region(nfull, tail // NS)

    return sc_sweep(vocab_lin, idsf2, upd2, gate2)


def kernel(decoder_hidden, encoder_outputs, vocab_dist, input_ids,
           W_att, b_att, W_v, W_gate, b_gate):
    B, S, H = encoder_outputs.shape
    V = vocab_dist.shape[1]
    wa = W_att[:, :H].T
    wb = W_att[:, H:].T
    ba = b_att.reshape(1, H)
    wv = W_v.reshape(1, H)
    wgh = W_gate[:, :H].T
    wgc = W_gate[:, H:].T
    bg = b_gate.reshape(1, 1)
    ids = input_ids.astype(jnp.int32)

    cw, gate, upd, idsf = _attn_call(
        decoder_hidden, encoder_outputs, ids, wa, wb, ba, wv, wgh, wgc, bg)
    vocab_lin = (vocab_dist.T.reshape(V // 8, 8, B // 128, 128)
                 .transpose(0, 2, 1, 3).reshape(B * V))
    flat = _sc_stream_call(vocab_lin,
                           idsf.reshape(B * SPAD // 128, 128),
                           upd.reshape(B * SPAD // 128, 128),
                           gate.reshape(B // 128, 128))
    final = (flat.reshape(V // 8, B // 128, 8, 128)
             .transpose(0, 2, 1, 3).reshape(V, B).T)
    return final, cw


# speedup vs baseline: 4.8941x; 4.8941x over previous
import functools

import jax
import jax.numpy as jnp
from jax import lax
from jax.experimental import pallas as pl
from jax.experimental.pallas import tpu as pltpu
from jax.experimental.pallas import tpu_sc as plsc
from jax._src.pallas import mpmd as _mpmd

SPAD = 256
NC, NS = 2, 16
NW = NC * NS


def _attn_call(dh, enc, ids, wa, wb, ba, wv, wgh, wgc, bg):
    B, S, H = enc.shape
    BB = 8

    def body(dh_ref, enc_ref, ids_ref, wa_ref, wb_ref, ba_ref, wv_ref,
             wgh_ref, wgc_ref, bg_ref,
             cw_ref, gate_ref, upd_ref, idsf_ref):
        pid = pl.program_id(0)
        dhb = dh_ref[...]
        encb = enc_ref[...]
        dpart = jnp.dot(dhb, wa_ref[...], preferred_element_type=jnp.float32)
        dpart = dpart + ba_ref[...]
        e = jnp.dot(encb.reshape(BB * S, H), wb_ref[...],
                    preferred_element_type=jnp.float32)
        e = jnp.tanh(e.reshape(BB, S, H) + dpart[:, None, :])
        sc = jnp.sum(e * wv_ref[...][None, :, :], axis=2)
        m = jnp.max(sc, axis=1, keepdims=True)
        ex = jnp.exp(sc - m)
        w = ex / jnp.sum(ex, axis=1, keepdims=True)
        cw_ref[...] = w
        ctx = jnp.sum(w[:, :, None] * encb, axis=1)
        g = jnp.dot(dhb, wgh_ref[...], preferred_element_type=jnp.float32)
        g = g + jnp.dot(ctx, wgc_ref[...], preferred_element_type=jnp.float32)
        g = jax.nn.sigmoid(g + bg_ref[...])
        gate_ref[...] = g
        idsb = ids_ref[...]
        ids_pad = jnp.concatenate(
            [idsb, jnp.zeros((BB, SPAD - S), jnp.int32)], axis=1)
        w_pad = jnp.concatenate(
            [w, jnp.zeros((BB, SPAD - S), jnp.float32)], axis=1)
        eq = (ids_pad[:, :, None] == ids_pad[:, None, :]).astype(jnp.float32)
        comb = jnp.sum(w_pad[:, :, None] * eq, axis=1)
        upd_ref[...] = comb * g
        row = pid * BB + lax.broadcasted_iota(jnp.int32, (BB, 1), 0)
        nbt = B // 128
        idsf_ref[...] = (((ids_pad >> 3) * nbt + (row >> 7)) << 10) \
            + ((ids_pad & 7) << 7) + (row & 127)

    return pl.pallas_call(
        body,
        grid=(B // BB,),
        in_specs=[
            pl.BlockSpec((BB, H), lambda i: (i, 0)),
            pl.BlockSpec((BB, S, H), lambda i: (i, 0, 0)),
            pl.BlockSpec((BB, S), lambda i: (i, 0)),
            pl.BlockSpec((H, H), lambda i: (0, 0)),
            pl.BlockSpec((H, H), lambda i: (0, 0)),
            pl.BlockSpec((1, H), lambda i: (0, 0)),
            pl.BlockSpec((1, H), lambda i: (0, 0)),
            pl.BlockSpec((H, 1), lambda i: (0, 0)),
            pl.BlockSpec((H, 1), lambda i: (0, 0)),
            pl.BlockSpec((1, 1), lambda i: (0, 0)),
        ],
        out_specs=[
            pl.BlockSpec((BB, S), lambda i: (i, 0)),
            pl.BlockSpec((BB, 1), lambda i: (i, 0)),
            pl.BlockSpec((BB, SPAD), lambda i: (i, 0)),
            pl.BlockSpec((BB, SPAD), lambda i: (i, 0)),
        ],
        out_shape=[
            jax.ShapeDtypeStruct((B, S), jnp.float32),
            jax.ShapeDtypeStruct((B, 1), jnp.float32),
            jax.ShapeDtypeStruct((B, SPAD), jnp.float32),
            jax.ShapeDtypeStruct((B, SPAD), jnp.int32),
        ],
    )(dh, enc, ids, wa, wb, ba, wv, wgh, wgc, bg)


def _scale_t(vocab_t, gate_row):
    V, B = vocab_t.shape
    RB = 1000 if V % 1000 == 0 else V

    def body(g_ref, v_ref, o_ref):
        z = (1.0 - g_ref[...]) * v_ref[...]
        zr = z.reshape(RB // 8, 8, B)
        for tc in range(B // 128):
            o_ref[:, tc, :, :] = zr[:, :, tc * 128:(tc + 1) * 128]

    return pl.pallas_call(
        body,
        grid=(V // RB,),
        in_specs=[
            pl.BlockSpec((1, B), lambda i: (0, 0)),
            pl.BlockSpec((RB, B), lambda i: (i, 0)),
        ],
        out_specs=pl.BlockSpec((RB // 8, B // 128, 8, 128),
                               lambda i: (i, 0, 0, 0)),
        out_shape=jax.ShapeDtypeStruct((V // 8, B // 128, 8, 128),
                                       jnp.float32),
    )(gate_row, vocab_t)


def _sc_rmw_call(final0_flat, idsf3, upd3):
    R = idsf3.shape[0]
    rows = R // NW
    mesh = plsc.VectorSubcoreMesh(core_axis_name="c", subcore_axis_name="s")

    def body(fin_in, idsf_ref, upd_ref, fout, idx_v, upd_v, old_v, gsem):
        del fin_in
        wid = lax.axis_index("s") * NC + lax.axis_index("c")
        base = wid * rows
        pltpu.sync_copy(idsf_ref.at[pl.ds(base, rows)], idx_v)
        pltpu.sync_copy(upd_ref.at[pl.ds(base, rows)], upd_v)

        def fire_gather(j, carry):
            pltpu.async_copy(fout.at[idx_v.at[j]], old_v.at[j], gsem)
            return carry

        lax.fori_loop(0, rows, fire_gather, 0)
        pltpu.make_async_copy(upd_ref.at[pl.ds(base, rows)], old_v, gsem).wait()

        def add_row(r, carry):
            for c in range(8):
                sl = pl.ds(c * 16, 16)
                old_v[r, sl] = old_v[r, sl] + upd_v[r, sl]
            return carry

        lax.fori_loop(0, rows, add_row, 0)

        def fire_scatter(j, carry):
            pltpu.async_copy(old_v.at[j], fout.at[idx_v.at[j]], gsem)
            return carry

        lax.fori_loop(0, rows, fire_scatter, 0)
        pltpu.make_async_copy(upd_ref.at[pl.ds(base, rows)], old_v, gsem).wait()

    run = _mpmd._mpmd_map(
        [(mesh, body)],
        out_types=jax.ShapeDtypeStruct(final0_flat.shape, jnp.float32),
        input_output_aliases={0: 0},
        scratch_types=[
            pltpu.VMEM((rows, 128), jnp.int32),
            pltpu.VMEM((rows, 128), jnp.float32),
            pltpu.VMEM((rows, 128), jnp.float32),
            pltpu.SemaphoreType.DMA,
        ],
    )
    return run(final0_flat, idsf3, upd3)


def kernel(decoder_hidden, encoder_outputs, vocab_dist, input_ids,
           W_att, b_att, W_v, W_gate, b_gate):
    B, S, H = encoder_outputs.shape
    V = vocab_dist.shape[1]
    wa = W_att[:, :H].T
    wb = W_att[:, H:].T
    ba = b_att.reshape(1, H)
    wv = W_v.reshape(1, H)
    wgh = W_gate[:, :H].T
    wgc = W_gate[:, H:].T
    bg = b_gate.reshape(1, 1)
    ids = input_ids.astype(jnp.int32)

    cw, gate, upd, idsf = _attn_call(
        decoder_hidden, encoder_outputs, ids, wa, wb, ba, wv, wgh, wgc, bg)
    P = _scale_t(vocab_dist.T, gate.reshape(1, B))
    flat = _sc_rmw_call(P.reshape(B * V),
                        idsf.reshape(B * SPAD // 128, 128),
                        upd.reshape(B * SPAD // 128, 128))
    final = (flat.reshape(V // 8, B // 128, 8, 128)
             .transpose(0, 2, 1, 3).reshape(V, B).T)
    return final, cw
